# baseline (speedup 1.0000x reference)
import functools

import jax
import jax.numpy as jnp
from jax import lax
from jax.experimental import pallas as pl
from jax.experimental.pallas import tpu as pltpu
from jax.experimental.pallas import tpu_sc as plsc

N = 10000
E = 320000
D_IN = 128
D_OUT = 128
C = 16

NC = 2
NS = 16
NW = NC * NS
CHUNK = 128
NCHUNK = E // CHUNK
CH_BASE = NCHUNK // NW
CH_EXTRA = NCHUNK % NW
PAIRS = (CH_BASE + CH_EXTRA + 1) // 2
ROWS_MAIN = 624
TAIL_BASE = NS * ROWS_MAIN
TAIL_ROWS = N - TAIL_BASE
SLOPE = 0.2

_EB = 32000


_EW = E // 8
_EWB = 4000


def _tc_prep_body(nf, w3, a_l, a_r, efw_in, bd, el, er, efw_out):
    i = pl.program_id(0)

    @pl.when(i == 0)
    def _():
        al_mat = jnp.sum(w3[...] * a_l[...][None], axis=-1)
        ar_mat = jnp.sum(w3[...] * a_r[...][None], axis=-1)
        nfv = nf[...]
        el[...] = jnp.dot(nfv, al_mat, preferred_element_type=jnp.float32)
        er[...] = jnp.dot(nfv, ar_mat, preferred_element_type=jnp.float32)

    efw_out[...] = jnp.dot(efw_in[...], bd[...],
                           preferred_element_type=jnp.float32)


def _tc_prep(node_feat, W3, al, ar, edge_feat_w, W_bd):
    return pl.pallas_call(
        _tc_prep_body,
        grid=(_EW // _EWB,),
        in_specs=[
            pl.BlockSpec((N, D_IN), lambda i: (0, 0)),
            pl.BlockSpec((D_IN, C, D_OUT), lambda i: (0, 0, 0)),
            pl.BlockSpec((C, D_OUT), lambda i: (0, 0)),
            pl.BlockSpec((C, D_OUT), lambda i: (0, 0)),
            pl.BlockSpec((_EWB, 128), lambda i: (i, 0)),
            pl.BlockSpec((128, 128), lambda i: (0, 0)),
        ],
        out_specs=[
            pl.BlockSpec((N, C), lambda i: (0, 0)),
            pl.BlockSpec((N, C), lambda i: (0, 0)),
            pl.BlockSpec((_EWB, 128), lambda i: (i, 0)),
        ],
        out_shape=[
            jax.ShapeDtypeStruct((N, C), jnp.float32),
            jax.ShapeDtypeStruct((N, C), jnp.float32),
            jax.ShapeDtypeStruct((_EW, 128), jnp.float32),
        ],
    )(node_feat, W3, al, ar, edge_feat_w, W_bd)


_mesh = plsc.VectorSubcoreMesh(core_axis_name="c", subcore_axis_name="s")


@functools.partial(
    pl.kernel,
    out_type=(
        jax.ShapeDtypeStruct((E, C), jnp.float32),
        jax.ShapeDtypeStruct((NC, N, C), jnp.float32),
    ),
    mesh=_mesh,
    compiler_params=pltpu.CompilerParams(use_tc_tiling_on_sc=False, needs_layout_passes=False),
    scratch_types=[
        pltpu.VMEM((CH_BASE + 1, CHUNK), jnp.int32),
        pltpu.VMEM((CH_BASE + 1, CHUNK), jnp.int32),
        pltpu.VMEM((2, CHUNK, C), jnp.float32),
        pltpu.VMEM((2, CHUNK, C), jnp.float32),
        pltpu.VMEM((2, CHUNK * C), jnp.float32),
        pltpu.VMEM((2, CHUNK, C), jnp.float32),
        pltpu.VMEM((ROWS_MAIN, C), jnp.float32),
        pltpu.VMEM_SHARED((N, C), jnp.float32),
        pltpu.SemaphoreType.DMA,
        pltpu.SemaphoreType.DMA,
        pltpu.SemaphoreType.DMA,
        pltpu.SemaphoreType.DMA,
    ],
)
def _sc_pass1(src_hbm, dst_hbm, el_hbm, er_hbm, ef_hbm,
              ex_hbm, spart_hbm,
              idx_s, idx_d, elb, erb, efb, exb, srow, s_sh,
              sem0, sem1, osem0, osem1):
    cid = lax.axis_index("c")
    sid = lax.axis_index("s")
    wid = sid * NC + cid
    start = CH_BASE * wid + jnp.minimum(wid, CH_EXTRA)
    n_w = CH_BASE + jnp.where(wid < CH_EXTRA, 1, 0)

    def zero_body(j, _):
        srow[j] = jnp.zeros((C,), jnp.float32)
        return 0

    lax.fori_loop(0, ROWS_MAIN, zero_body, 0, unroll=8)
    pltpu.sync_copy(srow, s_sh.at[pl.ds(sid * ROWS_MAIN, ROWS_MAIN)])

    @pl.when(sid == 0)
    def _():
        pltpu.sync_copy(srow.at[pl.ds(0, TAIL_ROWS)],
                        s_sh.at[pl.ds(TAIL_BASE, TAIL_ROWS)])

    pltpu.sync_copy(src_hbm.at[pl.ds(start, CH_BASE)], idx_s.at[pl.ds(0, CH_BASE)])
    pltpu.sync_copy(dst_hbm.at[pl.ds(start, CH_BASE)], idx_d.at[pl.ds(0, CH_BASE)])

    @pl.when(wid < CH_EXTRA)
    def _():
        pltpu.sync_copy(src_hbm.at[pl.ds(start + CH_BASE, 1)],
                        idx_s.at[pl.ds(CH_BASE, 1)])
        pltpu.sync_copy(dst_hbm.at[pl.ds(start + CH_BASE, 1)],
                        idx_d.at[pl.ds(CH_BASE, 1)])

    plsc.subcore_barrier()

    def issue(j, slot, sem):
        base = (start + j) * CHUNK
        pltpu.async_copy(el_hbm.at[idx_s.at[j]], elb.at[slot], sem)
        pltpu.async_copy(er_hbm.at[idx_d.at[j]], erb.at[slot], sem)
        pltpu.async_copy(ef_hbm.at[pl.ds(base * C, CHUNK * C)],
                         efb.at[slot], sem)

    def wait_in(slot, sem):
        pltpu.make_async_copy(el_hbm.at[idx_s.at[0]], elb.at[slot], sem).wait()
        pltpu.make_async_copy(er_hbm.at[idx_d.at[0]], erb.at[slot], sem).wait()
        pltpu.make_async_copy(ef_hbm.at[pl.ds(0, CHUNK * C)],
                              efb.at[slot], sem).wait()

    def drain_out(slot, osem):
        pltpu.make_async_copy(exb.at[slot], ex_hbm.at[pl.ds(0, CHUNK)],
                              osem).wait()

    def process(j, slot, osem):
        def row_body(r, _):
            v = elb[slot, r] + erb[slot, r]
            v = jnp.where(v > 0, v, SLOPE * v)
            efv = efb[slot, pl.ds(r * C, C)]
            exb[slot, r] = jnp.exp(v * efv)
            return 0

        lax.fori_loop(0, CHUNK, row_body, 0, unroll=16)
        pltpu.sync_copy(exb.at[slot], s_sh.at[idx_d.at[j]], add=True)
        pltpu.async_copy(exb.at[slot],
                         ex_hbm.at[pl.ds((start + j) * CHUNK, CHUNK)], osem)

    issue(0, 0, sem0)

    def pair_body(p, _):
        i0 = 2 * p
        i1 = i0 + 1

        @pl.when(i0 < n_w)
        def _():
            @pl.when(i1 < n_w)
            def _():
                issue(i1, 1, sem1)

            wait_in(0, sem0)

            @pl.when(p >= 1)
            def _():
                drain_out(0, osem0)

            process(i0, 0, osem0)

        @pl.when(i1 < n_w)
        def _():
            @pl.when(i1 + 1 < n_w)
            def _():
                issue(i1 + 1, 0, sem0)

            wait_in(1, sem1)

            @pl.when(p >= 1)
            def _():
                drain_out(1, osem1)

            process(i1, 1, osem1)

        return 0

    lax.fori_loop(0, PAIRS, pair_body, 0)
    drain_out(0, osem0)
    drain_out(1, osem1)
    plsc.subcore_barrier()

    pltpu.sync_copy(s_sh.at[pl.ds(sid * ROWS_MAIN, ROWS_MAIN)], srow)
    pltpu.sync_copy(srow, spart_hbm.at[cid, pl.ds(sid * ROWS_MAIN, ROWS_MAIN)])

    @pl.when(sid == 0)
    def _():
        pltpu.sync_copy(s_sh.at[pl.ds(TAIL_BASE, TAIL_ROWS)],
                        srow.at[pl.ds(0, TAIL_ROWS)])
        pltpu.sync_copy(srow.at[pl.ds(0, TAIL_ROWS)],
                        spart_hbm.at[cid, pl.ds(TAIL_BASE, TAIL_ROWS)])


@functools.partial(
    pl.kernel,
    out_type=jax.ShapeDtypeStruct((C, E), jnp.float32),
    mesh=_mesh,
    compiler_params=pltpu.CompilerParams(use_tc_tiling_on_sc=False, needs_layout_passes=False),
    scratch_types=[
        pltpu.VMEM((CH_BASE + 1, CHUNK), jnp.int32),
        pltpu.VMEM((2, CHUNK * C), jnp.float32),
        pltpu.VMEM((2, CHUNK, C), jnp.float32),
        pltpu.VMEM((2, CHUNK, C), jnp.float32),
        pltpu.VMEM((2, C, CHUNK), jnp.float32),
        pltpu.SemaphoreType.DMA,
        pltpu.SemaphoreType.DMA,
        pltpu.SemaphoreType.DMA,
        pltpu.SemaphoreType.DMA,
    ],
)
def _sc_pass2(dst_hbm, ex_hbm, s0_hbm, s1_hbm, out_hbm,
              idx_d, exb, s0b, s1b, outb, sem0, sem1, osem0, osem1):
    cid = lax.axis_index("c")
    sid = lax.axis_index("s")
    wid = sid * NC + cid
    start = CH_BASE * wid + jnp.minimum(wid, CH_EXTRA)
    n_w = CH_BASE + jnp.where(wid < CH_EXTRA, 1, 0)

    pltpu.sync_copy(dst_hbm.at[pl.ds(start, CH_BASE)], idx_d.at[pl.ds(0, CH_BASE)])

    @pl.when(wid < CH_EXTRA)
    def _():
        pltpu.sync_copy(dst_hbm.at[pl.ds(start + CH_BASE, 1)],
                        idx_d.at[pl.ds(CH_BASE, 1)])

    def issue(j, slot, sem):
        base = (start + j) * CHUNK
        pltpu.async_copy(s0_hbm.at[idx_d.at[j]], s0b.at[slot], sem)
        pltpu.async_copy(s1_hbm.at[idx_d.at[j]], s1b.at[slot], sem)
        pltpu.async_copy(ex_hbm.at[pl.ds(base * C, CHUNK * C)], exb.at[slot], sem)

    def wait_in(slot, sem):
        pltpu.make_async_copy(s0_hbm.at[idx_d.at[0]], s0b.at[slot], sem).wait()
        pltpu.make_async_copy(s1_hbm.at[idx_d.at[0]], s1b.at[slot], sem).wait()
        pltpu.make_async_copy(ex_hbm.at[pl.ds(0, CHUNK * C)], exb.at[slot], sem).wait()

    rows16 = lax.broadcasted_iota(jnp.int32, (C,), 0)

    def drain_out(slot, osem):
        pltpu.make_async_copy(outb.at[slot],
                              out_hbm.at[pl.ds(0, C), pl.ds(0, CHUNK)],
                              osem).wait()

    def process(j, slot, osem):
        def row_body(r, _):
            v = exb[slot, pl.ds(r * C, C)] / (s0b[slot, r] + s1b[slot, r])
            plsc.store_scatter(outb.at[slot],
                               [rows16, jnp.full((C,), r, jnp.int32)], v)
            return 0

        lax.fori_loop(0, CHUNK, row_body, 0, unroll=16)
        pltpu.async_copy(outb.at[slot],
                         out_hbm.at[pl.ds(0, C),
                                    pl.ds((start + j) * CHUNK, CHUNK)], osem)

    issue(0, 0, sem0)

    def pair_body(p, _):
        i0 = 2 * p
        i1 = i0 + 1

        @pl.when(i0 < n_w)
        def _():
            @pl.when(i1 < n_w)
            def _():
                issue(i1, 1, sem1)

            wait_in(0, sem0)

            @pl.when(p >= 1)
            def _():
                drain_out(0, osem0)

            process(i0, 0, osem0)

        @pl.when(i1 < n_w)
        def _():
            @pl.when(i1 + 1 < n_w)
            def _():
                issue(i1 + 1, 0, sem0)

            wait_in(1, sem1)

            @pl.when(p >= 1)
            def _():
                drain_out(1, osem1)

            process(i1, 1, osem1)

        return 0

    lax.fori_loop(0, PAIRS, pair_body, 0)
    drain_out(0, osem0)
    drain_out(1, osem1)


def kernel(node_feat, edge_index, edge_feat, W_fc, W_edge, attn_l, attn_r):
    src2d = edge_index[0].reshape(NCHUNK, CHUNK)
    dst2d = edge_index[1].reshape(NCHUNK, CHUNK)
    W3 = W_fc.reshape(D_IN, C, D_OUT)
    al = attn_l.reshape(C, D_OUT)
    ar = attn_r.reshape(C, D_OUT)
    W_bd = jnp.kron(jnp.eye(8, dtype=jnp.float32), W_edge)
    el, er, efw = _tc_prep(node_feat, W3, al, ar,
                           edge_feat.reshape(_EW, 128), W_bd)
    ex, spart = _sc_pass1(src2d, dst2d, el, er, efw.reshape(E * C))
    a_t = _sc_pass2(dst2d, ex.reshape(E * C), spart[0], spart[1])
    return a_t.T.reshape(E, C, 1)


# speedup vs baseline: 1.0305x; 1.0305x over previous
import functools

import jax
import jax.numpy as jnp
from jax import lax
from jax.experimental import pallas as pl
from jax.experimental.pallas import tpu as pltpu
from jax.experimental.pallas import tpu_sc as plsc

N = 10000
E = 320000
D_IN = 128
D_OUT = 128
C = 16

NC = 2
NS = 16
NW = NC * NS
CHUNK = 128
NCHUNK = E // CHUNK
CH_BASE = NCHUNK // NW
CH_EXTRA = NCHUNK % NW
PAIRS = (CH_BASE + CH_EXTRA + 1) // 2
ROWS_MAIN = 624
TAIL_BASE = NS * ROWS_MAIN
TAIL_ROWS = N - TAIL_BASE
SLOPE = 0.2

_EB = 32000


_EW = E // 8
_EWB = 4000


def _tc_prep_body(nf, w3, a_l, a_r, efw_in, bd, el, er, efw_out):
    i = pl.program_id(0)

    @pl.when(i == 0)
    def _():
        al_mat = jnp.sum(w3[...] * a_l[...][None], axis=-1)
        ar_mat = jnp.sum(w3[...] * a_r[...][None], axis=-1)
        nfv = nf[...]
        el[...] = jnp.dot(nfv, al_mat, preferred_element_type=jnp.float32)
        er[...] = jnp.dot(nfv, ar_mat, preferred_element_type=jnp.float32)

    efw_out[...] = jnp.dot(efw_in[...], bd[...],
                           preferred_element_type=jnp.float32)


def _tc_prep(node_feat, W3, al, ar, edge_feat_w, W_bd):
    return pl.pallas_call(
        _tc_prep_body,
        grid=(_EW // _EWB,),
        in_specs=[
            pl.BlockSpec((N, D_IN), lambda i: (0, 0)),
            pl.BlockSpec((D_IN, C, D_OUT), lambda i: (0, 0, 0)),
            pl.BlockSpec((C, D_OUT), lambda i: (0, 0)),
            pl.BlockSpec((C, D_OUT), lambda i: (0, 0)),
            pl.BlockSpec((_EWB, 128), lambda i: (i, 0)),
            pl.BlockSpec((128, 128), lambda i: (0, 0)),
        ],
        out_specs=[
            pl.BlockSpec((N, C), lambda i: (0, 0)),
            pl.BlockSpec((N, C), lambda i: (0, 0)),
            pl.BlockSpec((_EWB, 128), lambda i: (i, 0)),
        ],
        out_shape=[
            jax.ShapeDtypeStruct((N, C), jnp.float32),
            jax.ShapeDtypeStruct((N, C), jnp.float32),
            jax.ShapeDtypeStruct((_EW, 128), jnp.float32),
        ],
    )(node_feat, W3, al, ar, edge_feat_w, W_bd)


_mesh = plsc.VectorSubcoreMesh(core_axis_name="c", subcore_axis_name="s")


@functools.partial(
    pl.kernel,
    out_type=(
        jax.ShapeDtypeStruct((E, C), jnp.float32),
        jax.ShapeDtypeStruct((N, NC * C), jnp.float32),
    ),
    mesh=_mesh,
    compiler_params=pltpu.CompilerParams(use_tc_tiling_on_sc=False, needs_layout_passes=False),
    scratch_types=[
        pltpu.VMEM((CH_BASE + 1, CHUNK), jnp.int32),
        pltpu.VMEM((CH_BASE + 1, CHUNK), jnp.int32),
        pltpu.VMEM((2, CHUNK, C), jnp.float32),
        pltpu.VMEM((2, CHUNK, C), jnp.float32),
        pltpu.VMEM((2, CHUNK * C), jnp.float32),
        pltpu.VMEM((2, CHUNK, C), jnp.float32),
        pltpu.VMEM((ROWS_MAIN, C), jnp.float32),
        pltpu.VMEM_SHARED((N, C), jnp.float32),
        pltpu.SemaphoreType.DMA,
        pltpu.SemaphoreType.DMA,
        pltpu.SemaphoreType.DMA,
        pltpu.SemaphoreType.DMA,
    ],
)
def _sc_pass1(src_hbm, dst_hbm, el_hbm, er_hbm, ef_hbm,
              ex_hbm, spart_hbm,
              idx_s, idx_d, elb, erb, efb, exb, srow, s_sh,
              sem0, sem1, osem0, osem1):
    cid = lax.axis_index("c")
    sid = lax.axis_index("s")
    wid = sid * NC + cid
    start = CH_BASE * wid + jnp.minimum(wid, CH_EXTRA)
    n_w = CH_BASE + jnp.where(wid < CH_EXTRA, 1, 0)

    def zero_body(j, _):
        srow[j] = jnp.zeros((C,), jnp.float32)
        return 0

    lax.fori_loop(0, ROWS_MAIN, zero_body, 0, unroll=8)
    pltpu.sync_copy(srow, s_sh.at[pl.ds(sid * ROWS_MAIN, ROWS_MAIN)])

    @pl.when(sid == 0)
    def _():
        pltpu.sync_copy(srow.at[pl.ds(0, TAIL_ROWS)],
                        s_sh.at[pl.ds(TAIL_BASE, TAIL_ROWS)])

    pltpu.sync_copy(src_hbm.at[pl.ds(start, CH_BASE)], idx_s.at[pl.ds(0, CH_BASE)])
    pltpu.sync_copy(dst_hbm.at[pl.ds(start, CH_BASE)], idx_d.at[pl.ds(0, CH_BASE)])

    @pl.when(wid < CH_EXTRA)
    def _():
        pltpu.sync_copy(src_hbm.at[pl.ds(start + CH_BASE, 1)],
                        idx_s.at[pl.ds(CH_BASE, 1)])
        pltpu.sync_copy(dst_hbm.at[pl.ds(start + CH_BASE, 1)],
                        idx_d.at[pl.ds(CH_BASE, 1)])

    plsc.subcore_barrier()

    def issue(j, slot, sem):
        base = (start + j) * CHUNK
        pltpu.async_copy(el_hbm.at[idx_s.at[j]], elb.at[slot], sem)
        pltpu.async_copy(er_hbm.at[idx_d.at[j]], erb.at[slot], sem)
        pltpu.async_copy(ef_hbm.at[pl.ds(base * C, CHUNK * C)],
                         efb.at[slot], sem)

    def wait_in(slot, sem):
        pltpu.make_async_copy(el_hbm.at[idx_s.at[0]], elb.at[slot], sem).wait()
        pltpu.make_async_copy(er_hbm.at[idx_d.at[0]], erb.at[slot], sem).wait()
        pltpu.make_async_copy(ef_hbm.at[pl.ds(0, CHUNK * C)],
                              efb.at[slot], sem).wait()

    def drain_out(slot, osem):
        pltpu.make_async_copy(exb.at[slot], ex_hbm.at[pl.ds(0, CHUNK)],
                              osem).wait()

    def process(j, slot, osem):
        def row_body(r, _):
            v = elb[slot, r] + erb[slot, r]
            v = jnp.where(v > 0, v, SLOPE * v)
            efv = efb[slot, pl.ds(r * C, C)]
            exb[slot, r] = jnp.exp(v * efv)
            return 0

        lax.fori_loop(0, CHUNK, row_body, 0, unroll=16)
        pltpu.sync_copy(exb.at[slot], s_sh.at[idx_d.at[j]], add=True)
        pltpu.async_copy(exb.at[slot],
                         ex_hbm.at[pl.ds((start + j) * CHUNK, CHUNK)], osem)

    issue(0, 0, sem0)

    def pair_body(p, _):
        i0 = 2 * p
        i1 = i0 + 1

        @pl.when(i0 < n_w)
        def _():
            @pl.when(i1 < n_w)
            def _():
                issue(i1, 1, sem1)

            wait_in(0, sem0)

            @pl.when(p >= 1)
            def _():
                drain_out(0, osem0)

            process(i0, 0, osem0)

        @pl.when(i1 < n_w)
        def _():
            @pl.when(i1 + 1 < n_w)
            def _():
                issue(i1 + 1, 0, sem0)

            wait_in(1, sem1)

            @pl.when(p >= 1)
            def _():
                drain_out(1, osem1)

            process(i1, 1, osem1)

        return 0

    lax.fori_loop(0, PAIRS, pair_body, 0)
    drain_out(0, osem0)
    drain_out(1, osem1)
    plsc.subcore_barrier()

    pltpu.sync_copy(s_sh.at[pl.ds(sid * ROWS_MAIN, ROWS_MAIN)], srow)
    pltpu.sync_copy(srow, spart_hbm.at[pl.ds(sid * ROWS_MAIN, ROWS_MAIN),
                                       pl.ds(cid * C, C)])

    @pl.when(sid == 0)
    def _():
        pltpu.sync_copy(s_sh.at[pl.ds(TAIL_BASE, TAIL_ROWS)],
                        srow.at[pl.ds(0, TAIL_ROWS)])
        pltpu.sync_copy(srow.at[pl.ds(0, TAIL_ROWS)],
                        spart_hbm.at[pl.ds(TAIL_BASE, TAIL_ROWS),
                                     pl.ds(cid * C, C)])


@functools.partial(
    pl.kernel,
    out_type=jax.ShapeDtypeStruct((C, E), jnp.float32),
    mesh=_mesh,
    compiler_params=pltpu.CompilerParams(use_tc_tiling_on_sc=False, needs_layout_passes=False),
    scratch_types=[
        pltpu.VMEM((CH_BASE + 1, CHUNK), jnp.int32),
        pltpu.VMEM((2, CHUNK * C), jnp.float32),
        pltpu.VMEM((2, CHUNK, 2 * C), jnp.float32),
        pltpu.VMEM((2, C, CHUNK), jnp.float32),
        pltpu.SemaphoreType.DMA,
        pltpu.SemaphoreType.DMA,
        pltpu.SemaphoreType.DMA,
        pltpu.SemaphoreType.DMA,
    ],
)
def _sc_pass2(dst_hbm, ex_hbm, s01_hbm, out_hbm,
              idx_d, exb, s01b, outb, sem0, sem1, osem0, osem1):
    cid = lax.axis_index("c")
    sid = lax.axis_index("s")
    wid = sid * NC + cid
    start = CH_BASE * wid + jnp.minimum(wid, CH_EXTRA)
    n_w = CH_BASE + jnp.where(wid < CH_EXTRA, 1, 0)

    pltpu.sync_copy(dst_hbm.at[pl.ds(start, CH_BASE)], idx_d.at[pl.ds(0, CH_BASE)])

    @pl.when(wid < CH_EXTRA)
    def _():
        pltpu.sync_copy(dst_hbm.at[pl.ds(start + CH_BASE, 1)],
                        idx_d.at[pl.ds(CH_BASE, 1)])

    def issue(j, slot, sem):
        base = (start + j) * CHUNK
        pltpu.async_copy(s01_hbm.at[idx_d.at[j]], s01b.at[slot], sem)
        pltpu.async_copy(ex_hbm.at[pl.ds(base * C, CHUNK * C)], exb.at[slot], sem)

    def wait_in(slot, sem):
        pltpu.make_async_copy(s01_hbm.at[idx_d.at[0]], s01b.at[slot], sem).wait()
        pltpu.make_async_copy(ex_hbm.at[pl.ds(0, CHUNK * C)], exb.at[slot], sem).wait()

    rows16 = lax.broadcasted_iota(jnp.int32, (C,), 0)

    def drain_out(slot, osem):
        pltpu.make_async_copy(outb.at[slot],
                              out_hbm.at[pl.ds(0, C), pl.ds(0, CHUNK)],
                              osem).wait()

    def process(j, slot, osem):
        def row_body(r, _):
            sv = s01b[slot, r, pl.ds(0, C)] + s01b[slot, r, pl.ds(C, C)]
            v = exb[slot, pl.ds(r * C, C)] / sv
            plsc.store_scatter(outb.at[slot],
                               [rows16, jnp.full((C,), r, jnp.int32)], v)
            return 0

        lax.fori_loop(0, CHUNK, row_body, 0, unroll=16)
        pltpu.async_copy(outb.at[slot],
                         out_hbm.at[pl.ds(0, C),
                                    pl.ds((start + j) * CHUNK, CHUNK)], osem)

    issue(0, 0, sem0)

    def pair_body(p, _):
        i0 = 2 * p
        i1 = i0 + 1

        @pl.when(i0 < n_w)
        def _():
            @pl.when(i1 < n_w)
            def _():
                issue(i1, 1, sem1)

            wait_in(0, sem0)

            @pl.when(p >= 1)
            def _():
                drain_out(0, osem0)

            process(i0, 0, osem0)

        @pl.when(i1 < n_w)
        def _():
            @pl.when(i1 + 1 < n_w)
            def _():
                issue(i1 + 1, 0, sem0)

            wait_in(1, sem1)

            @pl.when(p >= 1)
            def _():
                drain_out(1, osem1)

            process(i1, 1, osem1)

        return 0

    lax.fori_loop(0, PAIRS, pair_body, 0)
    drain_out(0, osem0)
    drain_out(1, osem1)


def kernel(node_feat, edge_index, edge_feat, W_fc, W_edge, attn_l, attn_r):
    src2d = edge_index[0].reshape(NCHUNK, CHUNK)
    dst2d = edge_index[1].reshape(NCHUNK, CHUNK)
    W3 = W_fc.reshape(D_IN, C, D_OUT)
    al = attn_l.reshape(C, D_OUT)
    ar = attn_r.reshape(C, D_OUT)
    W_bd = jnp.kron(jnp.eye(8, dtype=jnp.float32), W_edge)
    el, er, efw = _tc_prep(node_feat, W3, al, ar,
                           edge_feat.reshape(_EW, 128), W_bd)
    ex, s01 = _sc_pass1(src2d, dst2d, el, er, efw.reshape(E * C))
    a_t = _sc_pass2(dst2d, ex.reshape(E * C), s01)
    return a_t.T.reshape(E, C, 1)
